# baseline (speedup 1.0000x reference)
import functools

import jax
import jax.numpy as jnp
from jax import lax
from jax.experimental import pallas as pl
from jax.experimental.pallas import tpu as pltpu
from jax.experimental.pallas import tpu_sc as plsc

VOCAB = 1000000
EMBED_DIM = 300
D_PAD = 304
TOTAL = 1024 * 10 * 20

_info = plsc.get_sparse_core_info()
_NC, _NS = _info.num_cores, _info.num_subcores
_NW = _NC * _NS
_PER_W = TOTAL // _NW
_CHUNK = 128
_NCHUNK = _PER_W // _CHUNK


def _gather_side(table_hbm, idx_hbm, out_hbm, idx_v, rows_v, sem, base):
    def body(i, carry):
        off = base + i * _CHUNK
        pltpu.sync_copy(idx_hbm.at[pl.ds(off, _CHUNK)], idx_v)
        pltpu.async_copy(table_hbm.at[idx_v], rows_v, sem).wait()
        pltpu.sync_copy(rows_v, out_hbm.at[pl.ds(off, _CHUNK)])
        return carry

    lax.fori_loop(0, _NCHUNK, body, 0)


@functools.partial(
    pl.kernel,
    out_type=(
        jax.ShapeDtypeStruct((TOTAL, D_PAD), jnp.float32),
        jax.ShapeDtypeStruct((TOTAL, D_PAD), jnp.float32),
    ),
    mesh=plsc.VectorSubcoreMesh(core_axis_name="c", subcore_axis_name="s"),
    compiler_params=pltpu.CompilerParams(use_tc_tiling_on_sc=False),
    scratch_types=[
        pltpu.VMEM((_CHUNK,), jnp.int32),
        pltpu.VMEM((_CHUNK, D_PAD), jnp.float32),
        pltpu.SemaphoreType.DMA,
    ],
)
def _embed_kernel(table_hbm, idx_l_hbm, idx_r_hbm, out_l_hbm, out_r_hbm,
                  idx_v, rows_v, sem):
    wid = lax.axis_index("s") * _NC + lax.axis_index("c")
    base = wid * _PER_W
    _gather_side(table_hbm, idx_l_hbm, out_l_hbm, idx_v, rows_v, sem, base)
    _gather_side(table_hbm, idx_r_hbm, out_r_hbm, idx_v, rows_v, sem, base)


def kernel(X_left, X_right, embed_weight):
    shp = X_left.shape
    idx_l = X_left.reshape(-1)
    idx_r = X_right.reshape(-1)
    tab_p = jnp.pad(embed_weight, ((0, 0), (0, D_PAD - EMBED_DIM)))
    out_l, out_r = _embed_kernel(tab_p, idx_l, idx_r)
    return (out_l[:, :EMBED_DIM].reshape(*shp, EMBED_DIM),
            out_r[:, :EMBED_DIM].reshape(*shp, EMBED_DIM))


# speedup vs baseline: 3.7634x; 3.7634x over previous
import functools

import jax
import jax.numpy as jnp
from jax import lax
from jax.experimental import pallas as pl
from jax.experimental.pallas import tpu as pltpu
from jax.experimental.pallas import tpu_sc as plsc

VOCAB = 1000000
EMBED_DIM = 300
D_PAD = 384
TOTAL = 1024 * 10 * 20

_info = plsc.get_sparse_core_info()
_NC, _NS = _info.num_cores, _info.num_subcores
_NW = _NC * _NS

_PER_W = TOTAL // _NW
_CHUNK = 128
_NCHUNK = _PER_W // _CHUNK


_VBLK = 2048
_NVBLK = -(-VOCAB // _VBLK)


def _transpose_body(tab_t_ref, tab_r_ref):
    x = tab_t_ref[...]
    xt = jnp.swapaxes(x, 0, 1)
    tab_r_ref[...] = jnp.pad(xt, ((0, 0), (0, D_PAD - EMBED_DIM)))


_transpose_kernel = pl.pallas_call(
    _transpose_body,
    out_shape=jax.ShapeDtypeStruct((VOCAB, D_PAD), jnp.float32),
    grid=(_NVBLK,),
    in_specs=[pl.BlockSpec((EMBED_DIM, _VBLK), lambda i: (0, i))],
    out_specs=pl.BlockSpec((_VBLK, D_PAD), lambda i: (i, 0)),
)


def _gather_side(table_hbm, idx_hbm, out_hbm, idx_v, rows_v, sem, base):
    def body(i, carry):
        off = base + i * _CHUNK
        pltpu.sync_copy(idx_hbm.at[pl.ds(off, _CHUNK)], idx_v)
        pltpu.async_copy(table_hbm.at[idx_v], rows_v, sem).wait()
        pltpu.sync_copy(rows_v, out_hbm.at[pl.ds(off, _CHUNK)])
        return carry

    lax.fori_loop(0, _NCHUNK, body, 0)


@functools.partial(
    pl.kernel,
    out_type=(
        jax.ShapeDtypeStruct((TOTAL, D_PAD), jnp.float32),
        jax.ShapeDtypeStruct((TOTAL, D_PAD), jnp.float32),
    ),
    mesh=plsc.VectorSubcoreMesh(core_axis_name="c", subcore_axis_name="s"),
    scratch_types=[
        pltpu.VMEM((_CHUNK,), jnp.int32),
        pltpu.VMEM((_CHUNK, D_PAD), jnp.float32),
        pltpu.SemaphoreType.DMA,
    ],
)
def _embed_kernel(table_hbm, idx_l_hbm, idx_r_hbm, out_l_hbm, out_r_hbm,
                  idx_v, rows_v, sem):
    wid = lax.axis_index("s") * _NC + lax.axis_index("c")
    base = wid * _PER_W
    _gather_side(table_hbm, idx_l_hbm, out_l_hbm, idx_v, rows_v, sem, base)
    _gather_side(table_hbm, idx_r_hbm, out_r_hbm, idx_v, rows_v, sem, base)


def kernel(X_left, X_right, embed_weight):
    shp = X_left.shape
    idx_l = X_left.reshape(-1)
    idx_r = X_right.reshape(-1)
    tab_t = embed_weight.T
    tab_r = _transpose_kernel(tab_t)
    out_l, out_r = _embed_kernel(tab_r, idx_l, idx_r)
    return (out_l[:, :EMBED_DIM].reshape(*shp, EMBED_DIM),
            out_r[:, :EMBED_DIM].reshape(*shp, EMBED_DIM))


# speedup vs baseline: 3.8833x; 1.0319x over previous
import functools

import jax
import jax.numpy as jnp
from jax import lax
from jax.experimental import pallas as pl
from jax.experimental.pallas import tpu as pltpu
from jax.experimental.pallas import tpu_sc as plsc

VOCAB = 1000000
EMBED_DIM = 300
D_PAD = 384
TOTAL = 1024 * 10 * 20

_info = plsc.get_sparse_core_info()
_NC, _NS = _info.num_cores, _info.num_subcores
_NW = _NC * _NS

_PER_W = TOTAL // _NW
_CHUNK = 128
_NCHUNK = _PER_W // _CHUNK


_VBLK = 4096
_NVBLK = -(-VOCAB // _VBLK)


def _transpose_body(tab_t_ref, tab_r_ref):
    x = tab_t_ref[...]
    xt = jnp.swapaxes(x, 0, 1)
    tab_r_ref[...] = jnp.pad(xt, ((0, 0), (0, D_PAD - EMBED_DIM)))


_transpose_kernel = pl.pallas_call(
    _transpose_body,
    out_shape=jax.ShapeDtypeStruct((VOCAB, D_PAD), jnp.float32),
    grid=(_NVBLK,),
    in_specs=[pl.BlockSpec((EMBED_DIM, _VBLK), lambda i: (0, i))],
    out_specs=pl.BlockSpec((_VBLK, D_PAD), lambda i: (i, 0)),
)


def _gather_side(table_hbm, idx_hbm, out_hbm, idx_v, rows_v, sem, base):
    def body(i, carry):
        off = base + i * _CHUNK
        pltpu.sync_copy(idx_hbm.at[pl.ds(off, _CHUNK)], idx_v)
        pltpu.async_copy(table_hbm.at[idx_v], rows_v, sem).wait()
        pltpu.sync_copy(rows_v, out_hbm.at[pl.ds(off, _CHUNK)])
        return carry

    lax.fori_loop(0, _NCHUNK, body, 0)


@functools.partial(
    pl.kernel,
    out_type=(
        jax.ShapeDtypeStruct((TOTAL, D_PAD), jnp.float32),
        jax.ShapeDtypeStruct((TOTAL, D_PAD), jnp.float32),
    ),
    mesh=plsc.VectorSubcoreMesh(core_axis_name="c", subcore_axis_name="s"),
    scratch_types=[
        pltpu.VMEM((_CHUNK,), jnp.int32),
        pltpu.VMEM((_CHUNK, D_PAD), jnp.float32),
        pltpu.SemaphoreType.DMA,
    ],
)
def _embed_kernel(table_hbm, idx_l_hbm, idx_r_hbm, out_l_hbm, out_r_hbm,
                  idx_v, rows_v, sem):
    wid = lax.axis_index("s") * _NC + lax.axis_index("c")
    base = wid * _PER_W
    _gather_side(table_hbm, idx_l_hbm, out_l_hbm, idx_v, rows_v, sem, base)
    _gather_side(table_hbm, idx_r_hbm, out_r_hbm, idx_v, rows_v, sem, base)


def kernel(X_left, X_right, embed_weight):
    shp = X_left.shape
    idx_l = X_left.reshape(-1)
    idx_r = X_right.reshape(-1)
    tab_t = embed_weight.T
    tab_r = _transpose_kernel(tab_t)
    out_l, out_r = _embed_kernel(tab_r, idx_l, idx_r)
    return (out_l[:, :EMBED_DIM].reshape(*shp, EMBED_DIM),
            out_r[:, :EMBED_DIM].reshape(*shp, EMBED_DIM))


# speedup vs baseline: 3.9058x; 1.0058x over previous
import functools

import jax
import jax.numpy as jnp
from jax import lax
from jax.experimental import pallas as pl
from jax.experimental.pallas import tpu as pltpu
from jax.experimental.pallas import tpu_sc as plsc

VOCAB = 1000000
EMBED_DIM = 300
D_PAD = 384
TOTAL = 1024 * 10 * 20

_info = plsc.get_sparse_core_info()
_NC, _NS = _info.num_cores, _info.num_subcores
_NW = _NC * _NS

_PER_W = TOTAL // _NW
_CHUNK = 128
_NCHUNK = _PER_W // _CHUNK


_VBLK = 8192
_NVBLK = -(-VOCAB // _VBLK)


def _transpose_body(tab_t_ref, tab_r_ref):
    x = tab_t_ref[...]
    xt = jnp.swapaxes(x, 0, 1)
    tab_r_ref[...] = jnp.pad(xt, ((0, 0), (0, D_PAD - EMBED_DIM)))


_transpose_kernel = pl.pallas_call(
    _transpose_body,
    out_shape=jax.ShapeDtypeStruct((VOCAB, D_PAD), jnp.float32),
    grid=(_NVBLK,),
    in_specs=[pl.BlockSpec((EMBED_DIM, _VBLK), lambda i: (0, i))],
    out_specs=pl.BlockSpec((_VBLK, D_PAD), lambda i: (i, 0)),
)


def _gather_side(table_hbm, idx_hbm, out_hbm, idx_v, rows_v, sem, base):
    def body(i, carry):
        off = base + i * _CHUNK
        pltpu.sync_copy(idx_hbm.at[pl.ds(off, _CHUNK)], idx_v)
        pltpu.async_copy(table_hbm.at[idx_v], rows_v, sem).wait()
        pltpu.sync_copy(rows_v, out_hbm.at[pl.ds(off, _CHUNK)])
        return carry

    lax.fori_loop(0, _NCHUNK, body, 0)


@functools.partial(
    pl.kernel,
    out_type=(
        jax.ShapeDtypeStruct((TOTAL, D_PAD), jnp.float32),
        jax.ShapeDtypeStruct((TOTAL, D_PAD), jnp.float32),
    ),
    mesh=plsc.VectorSubcoreMesh(core_axis_name="c", subcore_axis_name="s"),
    scratch_types=[
        pltpu.VMEM((_CHUNK,), jnp.int32),
        pltpu.VMEM((_CHUNK, D_PAD), jnp.float32),
        pltpu.SemaphoreType.DMA,
    ],
)
def _embed_kernel(table_hbm, idx_l_hbm, idx_r_hbm, out_l_hbm, out_r_hbm,
                  idx_v, rows_v, sem):
    wid = lax.axis_index("s") * _NC + lax.axis_index("c")
    base = wid * _PER_W
    _gather_side(table_hbm, idx_l_hbm, out_l_hbm, idx_v, rows_v, sem, base)
    _gather_side(table_hbm, idx_r_hbm, out_r_hbm, idx_v, rows_v, sem, base)


def kernel(X_left, X_right, embed_weight):
    shp = X_left.shape
    idx_l = X_left.reshape(-1)
    idx_r = X_right.reshape(-1)
    tab_t = embed_weight.T
    tab_r = _transpose_kernel(tab_t)
    out_l, out_r = _embed_kernel(tab_r, idx_l, idx_r)
    return (out_l[:, :EMBED_DIM].reshape(*shp, EMBED_DIM),
            out_r[:, :EMBED_DIM].reshape(*shp, EMBED_DIM))


# speedup vs baseline: 4.5053x; 1.1535x over previous
import functools

import jax
import jax.numpy as jnp
from jax import lax
from jax.experimental import pallas as pl
from jax.experimental.pallas import tpu as pltpu
from jax.experimental.pallas import tpu_sc as plsc

VOCAB = 1000000
EMBED_DIM = 300
D_PAD = 384
D_OUT = 304
B, NSENT, SLEN = 1024, 10, 20
NS_TOT = NSENT * SLEN
TOTAL = B * NS_TOT

_info = plsc.get_sparse_core_info()
_NC, _NS = _info.num_cores, _info.num_subcores
_NW = _NC * _NS

_VBLK = 4096
_NVBLK = -(-VOCAB // _VBLK)


def _table_transpose_body(tab_t_ref, tab_r_ref):
    x = tab_t_ref[...]
    xt = jnp.swapaxes(x, 0, 1)
    tab_r_ref[...] = jnp.pad(xt, ((0, 0), (0, D_PAD - EMBED_DIM)))


_table_transpose = pl.pallas_call(
    _table_transpose_body,
    out_shape=jax.ShapeDtypeStruct((VOCAB, D_PAD), jnp.float32),
    grid=(_NVBLK,),
    in_specs=[pl.BlockSpec((EMBED_DIM, _VBLK), lambda i: (0, i))],
    out_specs=pl.BlockSpec((_VBLK, D_PAD), lambda i: (i, 0)),
)

_PER_W = TOTAL // _NW
_CHUNK = 128
_NCHUNK = _PER_W // _CHUNK


@functools.partial(
    pl.kernel,
    out_type=jax.ShapeDtypeStruct((TOTAL, D_PAD), jnp.float32),
    mesh=plsc.VectorSubcoreMesh(core_axis_name="c", subcore_axis_name="s"),
    scratch_types=[
        pltpu.VMEM((_CHUNK,), jnp.int32),
        pltpu.VMEM((_CHUNK, D_PAD), jnp.float32),
        pltpu.SemaphoreType.DMA,
    ],
)
def _embed_gather(table_hbm, idx_hbm, out_hbm, idx_v, rows_v, sem):
    wid = lax.axis_index("s") * _NC + lax.axis_index("c")
    base = wid * _PER_W

    def body(i, carry):
        off = base + i * _CHUNK
        pltpu.sync_copy(idx_hbm.at[pl.ds(off, _CHUNK)], idx_v)
        pltpu.async_copy(table_hbm.at[idx_v], rows_v, sem).wait()
        pltpu.sync_copy(rows_v, out_hbm.at[pl.ds(off, _CHUNK)])
        return carry

    lax.fori_loop(0, _NCHUNK, body, 0)


def _out_transpose_body(rows_ref, out_ref):
    x = rows_ref[...]
    xt = jnp.swapaxes(x, 0, 1)
    out_ref[...] = xt[:D_OUT, :]


_out_transpose = pl.pallas_call(
    _out_transpose_body,
    out_shape=jax.ShapeDtypeStruct((NS_TOT * D_OUT, B), jnp.float32),
    grid=(NS_TOT,),
    in_specs=[pl.BlockSpec((B, D_PAD), lambda i: (i, 0))],
    out_specs=pl.BlockSpec((D_OUT, B), lambda i: (i, 0)),
)


def _lookup_side(tab_r, X):
    idx = X.transpose(1, 2, 0).reshape(-1)
    rows = _embed_gather(tab_r, idx)
    out_t = _out_transpose(rows)
    out4 = out_t.reshape(NSENT, SLEN, D_OUT, B)[:, :, :EMBED_DIM, :]
    return out4.transpose(3, 0, 1, 2)


def kernel(X_left, X_right, embed_weight):
    tab_t = embed_weight.T
    tab_r = _table_transpose(tab_t)
    return (_lookup_side(tab_r, X_left), _lookup_side(tab_r, X_right))


# speedup vs baseline: 4.5939x; 1.0197x over previous
import functools

import jax
import jax.numpy as jnp
from jax import lax
from jax.experimental import pallas as pl
from jax.experimental.pallas import tpu as pltpu
from jax.experimental.pallas import tpu_sc as plsc

VOCAB = 1000000
EMBED_DIM = 300
D_PAD = 384
D_OUT = 304
B, NSENT, SLEN = 1024, 10, 20
NS_TOT = NSENT * SLEN
TOTAL = B * NS_TOT

_info = plsc.get_sparse_core_info()
_NC, _NS = _info.num_cores, _info.num_subcores
_NW = _NC * _NS

_VBLK = 4096
_NVBLK = -(-VOCAB // _VBLK)


def _table_transpose_body(tab_t_ref, tab_r_ref):
    x = tab_t_ref[...]
    xt = jnp.swapaxes(x, 0, 1)
    tab_r_ref[...] = jnp.pad(xt, ((0, 0), (0, D_PAD - EMBED_DIM)))


_table_transpose = pl.pallas_call(
    _table_transpose_body,
    out_shape=jax.ShapeDtypeStruct((VOCAB, D_PAD), jnp.float32),
    grid=(_NVBLK,),
    in_specs=[pl.BlockSpec((EMBED_DIM, _VBLK), lambda i: (0, i))],
    out_specs=pl.BlockSpec((_VBLK, D_PAD), lambda i: (i, 0)),
)

_PER_W = TOTAL // _NW
_CHUNK = 128
_NCHUNK = _PER_W // _CHUNK


@functools.partial(
    pl.kernel,
    out_type=jax.ShapeDtypeStruct((TOTAL, D_PAD), jnp.float32),
    mesh=plsc.VectorSubcoreMesh(core_axis_name="c", subcore_axis_name="s"),
    scratch_types=[
        pltpu.VMEM((_PER_W,), jnp.int32),
        pltpu.VMEM((_CHUNK, D_PAD), jnp.float32),
        pltpu.VMEM((_CHUNK, D_PAD), jnp.float32),
        pltpu.SemaphoreType.DMA,
        pltpu.SemaphoreType.DMA,
    ],
)
def _embed_gather(table_hbm, idx_hbm, out_hbm, idx_v, rows0, rows1,
                  sem0, sem1):
    wid = lax.axis_index("s") * _NC + lax.axis_index("c")
    base = wid * _PER_W
    pltpu.sync_copy(idx_hbm.at[pl.ds(base, _PER_W)], idx_v)

    def _idx(i):
        off = pl.multiple_of(i * _CHUNK, _CHUNK)
        return idx_v.at[pl.ds(off, _CHUNK)]

    def start(i, rows, sem):
        pltpu.async_copy(table_hbm.at[_idx(i)], rows, sem)

    def finish(i, rows, sem):
        pltpu.make_async_copy(table_hbm.at[_idx(i)], rows, sem).wait()
        off = pl.multiple_of(base + i * _CHUNK, _CHUNK)
        pltpu.sync_copy(rows, out_hbm.at[pl.ds(off, _CHUNK)])

    start(0, rows0, sem0)

    def body(j, carry):
        i0 = 2 * j
        start(i0 + 1, rows1, sem1)
        finish(i0, rows0, sem0)

        @pl.when(j < _NCHUNK // 2 - 1)
        def _():
            start(i0 + 2, rows0, sem0)

        finish(i0 + 1, rows1, sem1)
        return carry

    lax.fori_loop(0, _NCHUNK // 2, body, 0)


def _out_transpose_body(rows_ref, out_ref):
    x = rows_ref[...]
    xt = jnp.swapaxes(x, 0, 1)
    out_ref[...] = xt[:D_OUT, :]


_out_transpose = pl.pallas_call(
    _out_transpose_body,
    out_shape=jax.ShapeDtypeStruct((NS_TOT * D_OUT, B), jnp.float32),
    grid=(NS_TOT,),
    in_specs=[pl.BlockSpec((B, D_PAD), lambda i: (i, 0))],
    out_specs=pl.BlockSpec((D_OUT, B), lambda i: (i, 0)),
)


def _lookup_side(tab_r, X):
    idx = X.transpose(1, 2, 0).reshape(-1)
    rows = _embed_gather(tab_r, idx)
    out_t = _out_transpose(rows)
    out4 = out_t.reshape(NSENT, SLEN, D_OUT, B)[:, :, :EMBED_DIM, :]
    return out4.transpose(3, 0, 1, 2)


def kernel(X_left, X_right, embed_weight):
    tab_t = embed_weight.T
    tab_r = _table_transpose(tab_t)
    return (_lookup_side(tab_r, X_left), _lookup_side(tab_r, X_right))


# speedup vs baseline: 4.6267x; 1.0071x over previous
import functools

import jax
import jax.numpy as jnp
from jax import lax
from jax.experimental import pallas as pl
from jax.experimental.pallas import tpu as pltpu
from jax.experimental.pallas import tpu_sc as plsc

VOCAB = 1000000
EMBED_DIM = 300
D_PAD = 384
D_OUT = 304
B, NSENT, SLEN = 1024, 10, 20
NS_TOT = NSENT * SLEN
TOTAL = B * NS_TOT

_info = plsc.get_sparse_core_info()
_NC, _NS = _info.num_cores, _info.num_subcores
_NW = _NC * _NS

_VBLK = 8192
_NVBLK = -(-VOCAB // _VBLK)


def _table_transpose_body(tab_t_ref, tab_r_ref):
    x = tab_t_ref[...]
    xt = jnp.swapaxes(x, 0, 1)
    tab_r_ref[...] = jnp.pad(xt, ((0, 0), (0, D_PAD - EMBED_DIM)))


_table_transpose = pl.pallas_call(
    _table_transpose_body,
    out_shape=jax.ShapeDtypeStruct((VOCAB, D_PAD), jnp.float32),
    grid=(_NVBLK,),
    in_specs=[pl.BlockSpec((EMBED_DIM, _VBLK), lambda i: (0, i))],
    out_specs=pl.BlockSpec((_VBLK, D_PAD), lambda i: (i, 0)),
)

_PER_W = TOTAL // _NW
_CHUNK = 128
_NCHUNK = _PER_W // _CHUNK


@functools.partial(
    pl.kernel,
    out_type=jax.ShapeDtypeStruct((TOTAL, D_PAD), jnp.float32),
    mesh=plsc.VectorSubcoreMesh(core_axis_name="c", subcore_axis_name="s"),
    scratch_types=[
        pltpu.VMEM((_PER_W,), jnp.int32),
        pltpu.VMEM((_CHUNK, D_PAD), jnp.float32),
        pltpu.VMEM((_CHUNK, D_PAD), jnp.float32),
        pltpu.SemaphoreType.DMA,
        pltpu.SemaphoreType.DMA,
    ],
)
def _embed_gather(table_hbm, idx_hbm, out_hbm, idx_v, rows0, rows1,
                  sem0, sem1):
    wid = lax.axis_index("s") * _NC + lax.axis_index("c")
    base = wid * _PER_W
    pltpu.sync_copy(idx_hbm.at[pl.ds(base, _PER_W)], idx_v)

    def _idx(i):
        off = pl.multiple_of(i * _CHUNK, _CHUNK)
        return idx_v.at[pl.ds(off, _CHUNK)]

    def start(i, rows, sem):
        pltpu.async_copy(table_hbm.at[_idx(i)], rows, sem)

    def finish(i, rows, sem):
        pltpu.make_async_copy(table_hbm.at[_idx(i)], rows, sem).wait()
        off = pl.multiple_of(base + i * _CHUNK, _CHUNK)
        pltpu.sync_copy(rows, out_hbm.at[pl.ds(off, _CHUNK)])

    start(0, rows0, sem0)

    def body(j, carry):
        i0 = 2 * j
        start(i0 + 1, rows1, sem1)
        finish(i0, rows0, sem0)

        @pl.when(j < _NCHUNK // 2 - 1)
        def _():
            start(i0 + 2, rows0, sem0)

        finish(i0 + 1, rows1, sem1)
        return carry

    lax.fori_loop(0, _NCHUNK // 2, body, 0)


def _out_transpose_body(rows_ref, out_ref):
    x = rows_ref[...]
    xt = jnp.swapaxes(x, 0, 1)
    out_ref[...] = xt[:D_OUT, :]


_out_transpose = pl.pallas_call(
    _out_transpose_body,
    out_shape=jax.ShapeDtypeStruct((NS_TOT * D_OUT, B), jnp.float32),
    grid=(NS_TOT,),
    in_specs=[pl.BlockSpec((B, D_PAD), lambda i: (i, 0))],
    out_specs=pl.BlockSpec((D_OUT, B), lambda i: (i, 0)),
)


def _finish_side(rows):
    out_t = _out_transpose(rows)
    out4 = out_t.reshape(NSENT, SLEN, D_OUT, B)[:, :, :EMBED_DIM, :]
    return out4.transpose(3, 0, 1, 2)


def kernel(X_left, X_right, embed_weight):
    tab_t = embed_weight.T
    tab_r = _table_transpose(tab_t)
    idx_l = X_left.transpose(1, 2, 0).reshape(-1)
    idx_r = X_right.transpose(1, 2, 0).reshape(-1)
    rows_l = _embed_gather(tab_r, idx_l)
    rows_r = _embed_gather(tab_r, idx_r)
    return (_finish_side(rows_l), _finish_side(rows_r))
